# baseline (speedup 1.0000x reference)
import functools

import jax
import jax.numpy as jnp
from jax import lax
from jax.experimental import pallas as pl
from jax.experimental.pallas import tpu as pltpu
from jax.experimental.pallas import tpu_sc as plsc

N = 10000
D = 128
NC = 2
NS = 16
NW = NC * NS
L = 16
NP = 10240
STRIPE = NP // NS

CH_A = 128
CH_C = 128
RB = 2
GA = 1
SA = 1
EB = 8
EA = 5

_MESH = plsc.VectorSubcoreMesh(
    core_axis_name="c", subcore_axis_name="s", num_cores=NC, num_subcores=NS)


def _zeros16():
  return jnp.zeros((L,), jnp.float32)


_SPLAT_DNUMS = lax.GatherDimensionNumbers(
    offset_dims=(), collapsed_slice_dims=(0,), start_index_map=(0,))


def _lane_splat(vec, r):
  idx = jnp.full((L, 1), r, jnp.int32)
  return lax.gather(vec, idx, _SPLAT_DNUMS, slice_sizes=(1,),
                    mode=lax.GatherScatterMode.PROMISE_IN_BOUNDS)


def _make_deg_kernel(ch):
  @functools.partial(
      pl.kernel,
      out_type=jax.ShapeDtypeStruct((NC, NP), jnp.float32),
      mesh=_MESH,
      scratch_types=[
          pltpu.VMEM((ch, CH_A), jnp.int32),
          pltpu.VMEM((ch, CH_A), jnp.float32),
          pltpu.VMEM((STRIPE,), jnp.float32),
          pltpu.SemaphoreType.DMA,
          pltpu.VMEM_SHARED((NP,), jnp.float32),
      ],
  )
  def deg_kernel(dst_hbm, ew_hbm, out_hbm, dst_v, ew_v, zbuf, sem, acc):
    cid = lax.axis_index("c")
    sid = lax.axis_index("s")
    wid = cid * NS + sid

    def zb(i, carry):
      zbuf[pl.ds(i * L, L)] = _zeros16()
      return carry
    lax.fori_loop(0, STRIPE // L, zb, 0)
    pltpu.sync_copy(zbuf, acc.at[pl.ds(sid * STRIPE, STRIPE)])
    pltpu.sync_copy(dst_hbm.at[pl.ds(wid * ch, ch)], dst_v)
    pltpu.sync_copy(ew_hbm.at[pl.ds(wid * ch, ch)], ew_v)
    plsc.subcore_barrier()

    grp = 8
    def body(g, carry):
      descs = []
      for j in range(grp):
        k = g * grp + j
        descs.append(pltpu.async_copy(
            ew_v.at[k], acc.at[dst_v.at[k]], sem, add=True))
      for d in descs:
        d.wait()
      return carry
    lax.fori_loop(0, ch // grp, body, 0)
    plsc.subcore_barrier()
    pltpu.sync_copy(acc.at[pl.ds(sid * STRIPE, STRIPE)],
                    out_hbm.at[cid, pl.ds(sid * STRIPE, STRIPE)])

  return deg_kernel


def _make_msg_kernel(ch):
  @functools.partial(
      pl.kernel,
      out_type=jax.ShapeDtypeStruct((NC, NP, D), jnp.float32),
      mesh=_MESH,
      scratch_types=[
          pltpu.VMEM((CH_C,), jnp.int32),
          pltpu.VMEM((CH_C,), jnp.int32),
          pltpu.VMEM((CH_C,), jnp.float32),
          pltpu.VMEM((CH_C, D), jnp.float32),
          pltpu.SemaphoreType.DMA,
          pltpu.VMEM_SHARED((NP, D), jnp.float32),
      ],
  )
  def msg_kernel(src_hbm, dst_hbm, ew_hbm, h1_hbm, out_hbm,
                 src_v, dst_v, ew_v, rows, sem, acc):
    cid = lax.axis_index("c")
    sid = lax.axis_index("s")
    wid = cid * NS + sid

    def zr(r, carry):
      for j in range(D // L):
        rows[r, pl.ds(j * L, L)] = _zeros16()
      return carry
    lax.fori_loop(0, CH_C, zr, 0)

    def zs(s, carry):
      pltpu.sync_copy(rows, acc.at[pl.ds(sid * STRIPE + s * CH_C, CH_C)])
      return carry
    lax.fori_loop(0, STRIPE // CH_C, zs, 0)
    plsc.subcore_barrier()

    def body(k, carry):
      base = (wid * ch + k) * CH_C
      pltpu.sync_copy(src_hbm.at[pl.ds(base, CH_C)], src_v)
      pltpu.sync_copy(dst_hbm.at[pl.ds(base, CH_C)], dst_v)
      pltpu.sync_copy(ew_hbm.at[pl.ds(base, CH_C)], ew_v)
      pltpu.async_copy(h1_hbm.at[src_v], rows, sem).wait()

      def scale16(e16, c2):
        ew16 = ew_v[pl.ds(e16 * L, L)]
        for r in range(L):
          srow = _lane_splat(ew16, r)
          row = e16 * L + r
          for j in range(D // L):
            rows[row, pl.ds(j * L, L)] = rows[row, pl.ds(j * L, L)] * srow
        return c2
      lax.fori_loop(0, CH_C // L, scale16, 0)

      pltpu.sync_copy(rows, acc.at[dst_v], add=True)
      return carry
    lax.fori_loop(0, ch, body, 0)
    plsc.subcore_barrier()

    pltpu.sync_copy(acc.at[pl.ds(sid * STRIPE, STRIPE)],
                    out_hbm.at[cid, pl.ds(sid * STRIPE, STRIPE)])

  return msg_kernel


_BR = 1024


def _tc_h1_body(x_ref, w_ref, d0_ref, d1_ref, h1_ref, dis_ref):
  deg = d0_ref[...] + d1_ref[...] + 1.0
  dis = lax.rsqrt(deg)
  h = jnp.dot(x_ref[...], w_ref[...], preferred_element_type=jnp.float32)
  h1_ref[...] = h * dis
  dis_ref[...] = dis


def _tc_h1(x_p, W, d0, d1):
  grid = (NP // _BR,)
  return pl.pallas_call(
      _tc_h1_body,
      grid=grid,
      in_specs=[
          pl.BlockSpec((_BR, D), lambda i: (i, 0)),
          pl.BlockSpec((D, D), lambda i: (0, 0)),
          pl.BlockSpec((_BR, 1), lambda i: (i, 0)),
          pl.BlockSpec((_BR, 1), lambda i: (i, 0)),
      ],
      out_specs=[
          pl.BlockSpec((_BR, D), lambda i: (i, 0)),
          pl.BlockSpec((_BR, 1), lambda i: (i, 0)),
      ],
      out_shape=[
          jax.ShapeDtypeStruct((NP, D), jnp.float32),
          jax.ShapeDtypeStruct((NP, 1), jnp.float32),
      ],
  )(x_p, W, d0, d1)


def _tc_head_body(a0_ref, a1_ref, h1_ref, dis_ref, b_ref, lw_ref, lb_ref,
                  o_ref):
  s = a0_ref[...] + a1_ref[...] + h1_ref[...]
  pre = s * dis_ref[...] + b_ref[...]
  z = jnp.maximum(pre, 0.0)
  y = jnp.dot(z, lw_ref[...], preferred_element_type=jnp.float32) + lb_ref[...]
  o_ref[...] = jax.nn.sigmoid(y)


def _tc_head(a0, a1, h1, dis, b2, lw, lb2):
  grid = (NP // _BR,)
  return pl.pallas_call(
      _tc_head_body,
      grid=grid,
      in_specs=[
          pl.BlockSpec((_BR, D), lambda i: (i, 0)),
          pl.BlockSpec((_BR, D), lambda i: (i, 0)),
          pl.BlockSpec((_BR, D), lambda i: (i, 0)),
          pl.BlockSpec((_BR, 1), lambda i: (i, 0)),
          pl.BlockSpec((1, D), lambda i: (0, 0)),
          pl.BlockSpec((D, 1), lambda i: (0, 0)),
          pl.BlockSpec((1, 1), lambda i: (0, 0)),
      ],
      out_specs=pl.BlockSpec((_BR, 1), lambda i: (i, 0)),
      out_shape=jax.ShapeDtypeStruct((NP, 1), jnp.float32),
  )(a0, a1, h1, dis, b2, lw, lb2)


@jax.jit
def kernel(x, edge_index, edge_weight, W, b, lin_w, lin_b):
  E = edge_weight.shape[0]
  ch_a = -(-E // (NW * CH_A))
  ch_a = -(-ch_a // 8) * 8
  EP = NW * ch_a * CH_A
  ch_c = EP // (NW * CH_C)
  pad = EP - E

  src = edge_index[0]
  dst = edge_index[1]
  if pad:
    src = jnp.concatenate([src, jnp.zeros((pad,), jnp.int32)])
    dst = jnp.concatenate([dst, jnp.zeros((pad,), jnp.int32)])
    ew = jnp.concatenate([edge_weight, jnp.zeros((pad,), jnp.float32)])
  else:
    ew = edge_weight
  x_p = jnp.pad(x, ((0, NP - N), (0, 0)))

  degp = _make_deg_kernel(ch_a)(
      dst.reshape(NW * ch_a, CH_A), ew.reshape(NW * ch_a, CH_A))
  d0 = degp[0][:, None]
  d1 = degp[1][:, None]
  h1, dis = _tc_h1(x_p, W, d0, d1)
  accp = _make_msg_kernel(ch_c)(src, dst, ew, h1)
  y = _tc_head(accp[0], accp[1], h1, dis,
               b.reshape(1, D), lin_w, lin_b.reshape(1, 1))
  return y[:N]


# speedup vs baseline: 1.3289x; 1.3289x over previous
import functools

import jax
import jax.numpy as jnp
from jax import lax
from jax.experimental import pallas as pl
from jax.experimental.pallas import tpu as pltpu
from jax.experimental.pallas import tpu_sc as plsc

N = 10000
D = 128
NC = 2
NS = 16
NW = NC * NS
L = 16
CHUNK = 128
NP = 10240
STRIPE = NP // NS

_MESH = plsc.VectorSubcoreMesh(
    core_axis_name="c", subcore_axis_name="s", num_cores=NC, num_subcores=NS)


def _zeros16():
  return jnp.zeros((L,), jnp.float32)


_SPLAT_DNUMS = lax.GatherDimensionNumbers(
    offset_dims=(), collapsed_slice_dims=(0,), start_index_map=(0,))


def _lane_splat(vec, r):
  idx = jnp.full((L, 1), r, jnp.int32)
  return lax.gather(vec, idx, _SPLAT_DNUMS, slice_sizes=(1,),
                    mode=lax.GatherScatterMode.PROMISE_IN_BOUNDS)


def _make_deg_kernel(ch):
  @functools.partial(
      pl.kernel,
      out_type=jax.ShapeDtypeStruct((NC, NP), jnp.float32),
      mesh=_MESH,
      scratch_types=[
          pltpu.VMEM((CHUNK,), jnp.int32),
          pltpu.VMEM((CHUNK,), jnp.float32),
          pltpu.VMEM((STRIPE,), jnp.float32),
          pltpu.VMEM_SHARED((NP,), jnp.float32),
      ],
  )
  def deg_kernel(dst_hbm, ew_hbm, out_hbm, dst_v, ew_v, zbuf, acc):
    cid = lax.axis_index("c")
    sid = lax.axis_index("s")
    wid = cid * NS + sid

    def zb(i, carry):
      zbuf[pl.ds(i * L, L)] = _zeros16()
      return carry
    lax.fori_loop(0, STRIPE // L, zb, 0)
    pltpu.sync_copy(zbuf, acc.at[pl.ds(sid * STRIPE, STRIPE)])
    plsc.subcore_barrier()

    def body(k, carry):
      base = (wid * ch + k) * CHUNK
      pltpu.sync_copy(dst_hbm.at[pl.ds(base, CHUNK)], dst_v)
      pltpu.sync_copy(ew_hbm.at[pl.ds(base, CHUNK)], ew_v)
      pltpu.sync_copy(ew_v, acc.at[dst_v], add=True)
      return carry
    lax.fori_loop(0, ch, body, 0)
    plsc.subcore_barrier()
    pltpu.sync_copy(acc.at[pl.ds(sid * STRIPE, STRIPE)],
                    out_hbm.at[cid, pl.ds(sid * STRIPE, STRIPE)])

  return deg_kernel


def _make_msg_kernel(ch):
  @functools.partial(
      pl.kernel,
      out_type=jax.ShapeDtypeStruct((NC, NP, D), jnp.float32),
      mesh=_MESH,
      scratch_types=[
          pltpu.VMEM((CHUNK,), jnp.int32),
          pltpu.VMEM((CHUNK,), jnp.int32),
          pltpu.VMEM((CHUNK,), jnp.float32),
          pltpu.VMEM((CHUNK, D), jnp.float32),
          pltpu.SemaphoreType.DMA,
          pltpu.VMEM_SHARED((NP, D), jnp.float32),
      ],
  )
  def msg_kernel(src_hbm, dst_hbm, ew_hbm, h1_hbm, out_hbm,
                 src_v, dst_v, ew_v, rows, sem, acc):
    cid = lax.axis_index("c")
    sid = lax.axis_index("s")
    wid = cid * NS + sid

    def zr(r, carry):
      for j in range(D // L):
        rows[r, pl.ds(j * L, L)] = _zeros16()
      return carry
    lax.fori_loop(0, CHUNK, zr, 0)

    def zs(s, carry):
      pltpu.sync_copy(rows, acc.at[pl.ds(sid * STRIPE + s * CHUNK, CHUNK)])
      return carry
    lax.fori_loop(0, STRIPE // CHUNK, zs, 0)
    plsc.subcore_barrier()

    def body(k, carry):
      base = (wid * ch + k) * CHUNK
      pltpu.sync_copy(src_hbm.at[pl.ds(base, CHUNK)], src_v)
      pltpu.sync_copy(dst_hbm.at[pl.ds(base, CHUNK)], dst_v)
      pltpu.sync_copy(ew_hbm.at[pl.ds(base, CHUNK)], ew_v)
      pltpu.async_copy(h1_hbm.at[src_v], rows, sem).wait()

      def scale(e16, c2):
        ew16 = ew_v[pl.ds(e16 * L, L)]
        for r in range(L):
          srow = _lane_splat(ew16, r)
          row = e16 * L + r
          for j in range(D // L):
            rows[row, pl.ds(j * L, L)] = rows[row, pl.ds(j * L, L)] * srow
        return c2
      lax.fori_loop(0, CHUNK // L, scale, 0)

      pltpu.sync_copy(rows, acc.at[dst_v], add=True)
      return carry
    lax.fori_loop(0, ch, body, 0)
    plsc.subcore_barrier()

    pltpu.sync_copy(acc.at[pl.ds(sid * STRIPE, STRIPE)],
                    out_hbm.at[cid, pl.ds(sid * STRIPE, STRIPE)])

  return msg_kernel


_BR = 1024


def _tc_h1_body(x_ref, w_ref, d0_ref, d1_ref, h1_ref, dis_ref):
  deg = d0_ref[...] + d1_ref[...] + 1.0
  dis = lax.rsqrt(deg)
  h = jnp.dot(x_ref[...], w_ref[...], preferred_element_type=jnp.float32)
  h1_ref[...] = h * dis
  dis_ref[...] = dis


def _tc_h1(x_p, W, d0, d1):
  grid = (NP // _BR,)
  return pl.pallas_call(
      _tc_h1_body,
      grid=grid,
      in_specs=[
          pl.BlockSpec((_BR, D), lambda i: (i, 0)),
          pl.BlockSpec((D, D), lambda i: (0, 0)),
          pl.BlockSpec((_BR, 1), lambda i: (i, 0)),
          pl.BlockSpec((_BR, 1), lambda i: (i, 0)),
      ],
      out_specs=[
          pl.BlockSpec((_BR, D), lambda i: (i, 0)),
          pl.BlockSpec((_BR, 1), lambda i: (i, 0)),
      ],
      out_shape=[
          jax.ShapeDtypeStruct((NP, D), jnp.float32),
          jax.ShapeDtypeStruct((NP, 1), jnp.float32),
      ],
  )(x_p, W, d0, d1)


def _tc_head_body(a0_ref, a1_ref, h1_ref, dis_ref, b_ref, lw_ref, lb_ref,
                  o_ref):
  s = a0_ref[...] + a1_ref[...] + h1_ref[...]
  pre = s * dis_ref[...] + b_ref[...]
  z = jnp.maximum(pre, 0.0)
  y = jnp.dot(z, lw_ref[...], preferred_element_type=jnp.float32) + lb_ref[...]
  o_ref[...] = jax.nn.sigmoid(y)


def _tc_head(a0, a1, h1, dis, b2, lw, lb2):
  grid = (NP // _BR,)
  return pl.pallas_call(
      _tc_head_body,
      grid=grid,
      in_specs=[
          pl.BlockSpec((_BR, D), lambda i: (i, 0)),
          pl.BlockSpec((_BR, D), lambda i: (i, 0)),
          pl.BlockSpec((_BR, D), lambda i: (i, 0)),
          pl.BlockSpec((_BR, 1), lambda i: (i, 0)),
          pl.BlockSpec((1, D), lambda i: (0, 0)),
          pl.BlockSpec((D, 1), lambda i: (0, 0)),
          pl.BlockSpec((1, 1), lambda i: (0, 0)),
      ],
      out_specs=pl.BlockSpec((_BR, 1), lambda i: (i, 0)),
      out_shape=jax.ShapeDtypeStruct((NP, 1), jnp.float32),
  )(a0, a1, h1, dis, b2, lw, lb2)


@jax.jit
def kernel(x, edge_index, edge_weight, W, b, lin_w, lin_b):
  E = edge_weight.shape[0]
  ch = -(-E // (NW * CHUNK))
  EP = NW * ch * CHUNK
  pad = EP - E

  src = edge_index[0]
  dst = edge_index[1]
  if pad:
    src = jnp.concatenate([src, jnp.zeros((pad,), jnp.int32)])
    dst = jnp.concatenate([dst, jnp.zeros((pad,), jnp.int32)])
    ew = jnp.concatenate([edge_weight, jnp.zeros((pad,), jnp.float32)])
  else:
    ew = edge_weight
  x_p = jnp.pad(x, ((0, NP - N), (0, 0)))

  degp = _make_deg_kernel(ch)(dst, ew)
  d0 = degp[0][:, None]
  d1 = degp[1][:, None]
  h1, dis = _tc_h1(x_p, W, d0, d1)
  accp = _make_msg_kernel(ch)(src, dst, ew, h1)
  y = _tc_head(accp[0], accp[1], h1, dis,
               b.reshape(1, D), lin_w, lin_b.reshape(1, 1))
  return y[:N]
